# Initial kernel scaffold; baseline (speedup 1.0000x reference)
import functools

import jax
import jax.numpy as jnp
from jax import lax
from jax.experimental import pallas as pl
from jax.experimental.pallas import tpu as pltpu
from jax.experimental.pallas import tpu_sc as plsc

_V = 100000
_D = 128
_B = 16384
_XMAX = 100.0
_ALPHA = 0.75

_NC = 2
_NS = 16
_L = 16
_NW = _NC * _NS
_BPW = _B // _NW
_CH = 256
_NCH = _BPW // _CH

_LN2 = 0.6931471805599453
_LN_XMAX = 4.605170185988092
_SQRT2 = 1.4142135623730951


def _ln16(v):
    bits = plsc.bitcast(v, jnp.int32)
    e = ((bits >> 23) & 0xFF) - 127
    m = plsc.bitcast((bits & 0x7FFFFF) | 0x3F800000, jnp.float32)
    big = m > _SQRT2
    m = jnp.where(big, m * 0.5, m)
    ef = (e + jnp.where(big, 1, 0)).astype(jnp.float32)
    s = (m - 1.0) / (m + 1.0)
    s2 = s * s
    p = 1.0 + s2 * ((1.0 / 3.0) + s2 * ((1.0 / 5.0) + s2 * (1.0 / 7.0)))
    return ef * _LN2 + 2.0 * s * p


@functools.partial(
    pl.kernel,
    out_type=jax.ShapeDtypeStruct((_NW, _L), jnp.float32),
    mesh=plsc.VectorSubcoreMesh(core_axis_name="c", subcore_axis_name="s"),
    scratch_types=[
        pltpu.VMEM((_CH,), jnp.int32),
        pltpu.VMEM((_CH,), jnp.int32),
        pltpu.VMEM((_CH,), jnp.float32),
        pltpu.VMEM((_CH, _D), jnp.float32),
        pltpu.VMEM((_CH, _D), jnp.float32),
        pltpu.VMEM((_CH,), jnp.float32),
        pltpu.VMEM((_CH,), jnp.float32),
        pltpu.VMEM((_L,), jnp.float32),
        pltpu.SemaphoreType.DMA,
    ],
)
def _glove_sc(i_hbm, j_hbm, x_hbm, w_hbm, wt_hbm, b_hbm, bt_hbm, out_hbm,
              iv, jv, xv, wiv, wjv, biv, bjv, accv, sem):
    wid = lax.axis_index("s") * _NC + lax.axis_index("c")
    base = wid * _BPW
    acc = jnp.zeros((_L,), jnp.float32)
    for c in range(_NCH):
        off = base + c * _CH
        pltpu.sync_copy(i_hbm.at[pl.ds(off, _CH)], iv)
        pltpu.sync_copy(j_hbm.at[pl.ds(off, _CH)], jv)
        pltpu.sync_copy(x_hbm.at[pl.ds(off, _CH)], xv)
        cp1 = pltpu.async_copy(w_hbm.at[iv], wiv, sem)
        cp2 = pltpu.async_copy(wt_hbm.at[jv], wjv, sem)
        cp3 = pltpu.async_copy(b_hbm.at[iv], biv, sem)
        cp4 = pltpu.async_copy(bt_hbm.at[jv], bjv, sem)
        cp1.wait()
        cp2.wait()
        cp3.wait()
        cp4.wait()

        def gbody(g, acc):
            rows = lax.iota(jnp.int32, _L) + g * _L

            def dbody(d, dots):
                dcol = jnp.full((_L,), d, jnp.int32)
                a = plsc.load_gather(wiv, [rows, dcol])
                b = plsc.load_gather(wjv, [rows, dcol])
                return dots + a * b

            dots = lax.fori_loop(0, _D, dbody, jnp.zeros((_L,), jnp.float32))
            sl = pl.ds(g * _L, _L)
            lnx = _ln16(xv[sl])
            f = jnp.minimum(jnp.exp(_ALPHA * (lnx - _LN_XMAX)), 1.0)
            diff = dots + biv[sl] + bjv[sl] - lnx
            return acc + f * diff * diff

        acc = lax.fori_loop(0, _CH // _L, gbody, acc)
    accv[...] = acc
    pltpu.sync_copy(accv, out_hbm.at[wid])


def kernel(i, j, x, weight, weight_tilde, bias, bias_tilde):
    parts = _glove_sc(i, j, x, weight, weight_tilde, bias, bias_tilde)
    return jnp.sum(parts) / _B


# speedup vs baseline: 1.2029x; 1.2029x over previous
import functools

import jax
import jax.numpy as jnp
from jax import lax
from jax.experimental import pallas as pl
from jax.experimental.pallas import tpu as pltpu
from jax.experimental.pallas import tpu_sc as plsc

_V = 100000
_D = 128
_B = 16384
_XMAX = 100.0
_ALPHA = 0.75

_NC = 2
_NS = 16
_L = 16
_NW = _NC * _NS
_BPW = _B // _NW
_CH = 256
_NCH = _BPW // _CH

_LN2 = 0.6931471805599453
_LN_XMAX = 4.605170185988092
_SQRT2 = 1.4142135623730951


def _ln16(v):
    bits = plsc.bitcast(v, jnp.int32)
    e = ((bits >> 23) & 0xFF) - 127
    m = plsc.bitcast((bits & 0x7FFFFF) | 0x3F800000, jnp.float32)
    big = m > _SQRT2
    m = jnp.where(big, m * 0.5, m)
    ef = (e + jnp.where(big, 1, 0)).astype(jnp.float32)
    s = (m - 1.0) / (m + 1.0)
    s2 = s * s
    p = 1.0 + s2 * ((1.0 / 3.0) + s2 * ((1.0 / 5.0) + s2 * (1.0 / 7.0)))
    return ef * _LN2 + 2.0 * s * p


@functools.partial(
    pl.kernel,
    out_type=jax.ShapeDtypeStruct((_NW, _L), jnp.float32),
    mesh=plsc.VectorSubcoreMesh(core_axis_name="c", subcore_axis_name="s"),
    compiler_params=pltpu.CompilerParams(needs_layout_passes=False),
    scratch_types=[
        pltpu.VMEM((_CH,), jnp.int32),
        pltpu.VMEM((_CH,), jnp.int32),
        pltpu.VMEM((_CH,), jnp.float32),
        pltpu.VMEM((_CH, _D), jnp.float32),
        pltpu.VMEM((_CH, _D), jnp.float32),
        pltpu.VMEM((_CH,), jnp.float32),
        pltpu.VMEM((_CH,), jnp.float32),
        pltpu.VMEM((_L,), jnp.float32),
        pltpu.SemaphoreType.DMA,
    ],
)
def _glove_sc(i_hbm, j_hbm, x_hbm, w_hbm, wt_hbm, b_hbm, bt_hbm, out_hbm,
              iv, jv, xv, wiv, wjv, biv, bjv, accv, sem):
    wid = lax.axis_index("s") * _NC + lax.axis_index("c")
    base = wid * _BPW
    acc = jnp.zeros((_L,), jnp.float32)
    for c in range(_NCH):
        off = base + c * _CH
        pltpu.sync_copy(i_hbm.at[pl.ds(off, _CH)], iv)
        pltpu.sync_copy(j_hbm.at[pl.ds(off, _CH)], jv)
        pltpu.sync_copy(x_hbm.at[pl.ds(off, _CH)], xv)
        cp1 = pltpu.async_copy(w_hbm.at[iv], wiv, sem)
        cp2 = pltpu.async_copy(wt_hbm.at[jv], wjv, sem)
        cp3 = pltpu.async_copy(b_hbm.at[iv], biv, sem)
        cp4 = pltpu.async_copy(bt_hbm.at[jv], bjv, sem)
        cp1.wait()
        cp2.wait()
        cp3.wait()
        cp4.wait()

        def gbody(g, acc):
            lane = lax.iota(jnp.int32, _L)
            dots = jnp.zeros((_L,), jnp.float32)
            for l in range(_L):
                p = g * _L + l
                s = jnp.zeros((_L,), jnp.float32)
                for k in range(_D // _L):
                    s = s + wiv[p, pl.ds(k * _L, _L)] * wjv[p, pl.ds(k * _L, _L)]
                dots = jnp.where(lane == l, jnp.sum(s), dots)
            sl = pl.ds(g * _L, _L)
            lnx = _ln16(xv[sl])
            f = jnp.minimum(jnp.exp(_ALPHA * (lnx - _LN_XMAX)), 1.0)
            diff = dots + biv[sl] + bjv[sl] - lnx
            return acc + f * diff * diff

        acc = lax.fori_loop(0, _CH // _L, gbody, acc)
    accv[...] = acc
    pltpu.sync_copy(accv, out_hbm.at[wid])


def kernel(i, j, x, weight, weight_tilde, bias, bias_tilde):
    parts = _glove_sc(i, j, x, weight, weight_tilde, bias, bias_tilde)
    return jnp.sum(parts) / _B
